# Initial kernel scaffold; baseline (speedup 1.0000x reference)
import jax
import jax.numpy as jnp
from jax.experimental import pallas as pl


def kernel(logits, targets):
    raise NotImplementedError("write your pallas kernel here")


# speedup vs baseline: 1.1019x; 1.1019x over previous
import jax
import jax.numpy as jnp
from jax.experimental import pallas as pl

_B, _N = 128, 32768
_CLIP = 0.05
_ROWS_PER_BLOCK = 16


def _loss_block_kernel(x_ref, t_ref, out_ref):
    x = x_ref[...]
    t = t_ref[...]
    s = jax.nn.sigmoid(x)
    pos_arg = jnp.maximum(s, 1e-8)
    neg_arg = jnp.minimum((1.0 + _CLIP) - s, 1.0)
    arg = jnp.where(t == 1, pos_arg, neg_arg)
    part = jnp.sum(jnp.log(arg)).reshape(1, 1)

    @pl.when(pl.program_id(0) == 0)
    def _init():
        out_ref[...] = jnp.zeros((1, 1), jnp.float32)

    out_ref[...] += -part


def kernel(logits, targets):
    grid = _B // _ROWS_PER_BLOCK
    out = pl.pallas_call(
        _loss_block_kernel,
        grid=(grid,),
        in_specs=[
            pl.BlockSpec((_ROWS_PER_BLOCK, _N), lambda i: (i, 0)),
            pl.BlockSpec((_ROWS_PER_BLOCK, _N), lambda i: (i, 0)),
        ],
        out_specs=pl.BlockSpec((1, 1), lambda i: (0, 0)),
        out_shape=jax.ShapeDtypeStruct((1, 1), jnp.float32),
    )(logits, targets)
    return out[0, 0]


# speedup vs baseline: 1.1850x; 1.0754x over previous
import jax
import jax.numpy as jnp
from jax.experimental import pallas as pl

_B, _N = 128, 32768
_CLIP = 0.05
_ROWS_PER_BLOCK = 32


def _loss_block_kernel(x_ref, t_ref, out_ref):
    x = x_ref[...]
    t = t_ref[...]
    s = jax.nn.sigmoid(x)
    pos_arg = jnp.maximum(s, 1e-8)
    neg_arg = jnp.minimum((1.0 + _CLIP) - s, 1.0)
    arg = jnp.where(t == 1, pos_arg, neg_arg)
    part = jnp.sum(jnp.log(arg)).reshape(1, 1)

    @pl.when(pl.program_id(0) == 0)
    def _init():
        out_ref[...] = jnp.zeros((1, 1), jnp.float32)

    out_ref[...] += -part


def kernel(logits, targets):
    grid = _B // _ROWS_PER_BLOCK
    out = pl.pallas_call(
        _loss_block_kernel,
        grid=(grid,),
        in_specs=[
            pl.BlockSpec((_ROWS_PER_BLOCK, _N), lambda i: (i, 0)),
            pl.BlockSpec((_ROWS_PER_BLOCK, _N), lambda i: (i, 0)),
        ],
        out_specs=pl.BlockSpec((1, 1), lambda i: (0, 0)),
        out_shape=jax.ShapeDtypeStruct((1, 1), jnp.float32),
    )(logits, targets)
    return out[0, 0]
